# Initial kernel scaffold; baseline (speedup 1.0000x reference)
import jax
import jax.numpy as jnp
from jax.experimental import pallas as pl


def kernel(entity_emb_sr, entity_emb_tg, rel_emb_sr, rel_emb_tg, W0, W1, edge_index_sr, edge_index_tg, sr_data, tg_data, sr_rel_data, tg_rel_data):
    raise NotImplementedError("write your pallas kernel here")


# speedup vs baseline: 8.1830x; 8.1830x over previous
import jax
import jax.numpy as jnp
from jax import lax
from jax.experimental import pallas as pl
from jax.experimental.pallas import tpu as pltpu
from jax.experimental.pallas import tpu_sc as plsc

_N = 10000
_D = 128
_E = 320000
_R = 1000
_B = 4096
_C = 25
_NC = 2
_NS = 16
_NW = _NC * _NS

_DH = 64
_NPAD = 10240
_PT = _NPAD // _NS
_EK = 80
_ECH = _E // _NS // _EK
_ROWS_T = _NPAD // _NS
_ZR = 128
_LB = 1000
_GB = _N // _LB
_LKC = _B * _C // _NW // _D


def _sc_mesh():
    return plsc.VectorSubcoreMesh(
        core_axis_name="c", subcore_axis_name="s",
        num_cores=_NC, num_subcores=_NS)


def _norm_body(dst_hbm, norm_hbm, histv, dstv, redv, normv, shared):
    c = lax.axis_index("c")
    s = lax.axis_index("s")
    zero16 = jnp.zeros((16,), jnp.float32)
    ones16 = jnp.ones((16,), jnp.float32)

    def zb(i, carry):
        histv[pl.ds(i * 16, 16)] = zero16
        return carry
    lax.fori_loop(0, _NPAD // 16, zb, 0)

    pltpu.sync_copy(dst_hbm.at[c, s], dstv)

    def hb(i, carry):
        for j in range(_EK // 16):
            idx = dstv[i, pl.ds(j * 16, 16)]
            plsc.addupdate_scatter(histv, [idx], ones16)
        return carry
    lax.fori_loop(0, _ECH, hb, 0)

    pltpu.sync_copy(histv, shared.at[s])
    plsc.subcore_barrier()

    base = s * _PT
    for r in range(_NS):
        pltpu.sync_copy(shared.at[r, pl.ds(base, _PT)], redv.at[r])

    def nb(v, carry):
        d = redv[0, pl.ds(v * 16, 16)]
        for r in range(1, _NS):
            d = d + redv[r, pl.ds(v * 16, 16)]
        d = d + 1.0
        bits = plsc.bitcast(d, jnp.int32)
        bits = jnp.int32(0x5F3759DF) - (bits >> 1)
        y = plsc.bitcast(bits, jnp.float32)
        for _ in range(3):
            y = y * (1.5 - 0.5 * d * y * y)
        normv[pl.ds(v * 16, 16)] = y
        return carry
    lax.fori_loop(0, _PT // 16, nb, 0)

    pltpu.sync_copy(normv, norm_hbm.at[c, pl.ds(base, _PT)])


def _norm_call(dst_r):
    f = pl.kernel(
        _norm_body,
        out_type=jax.ShapeDtypeStruct((_NC, _NPAD), jnp.float32),
        mesh=_sc_mesh(),
        scratch_types=[
            pltpu.VMEM((_NPAD,), jnp.float32),
            pltpu.VMEM((_ECH, _EK), jnp.int32),
            pltpu.VMEM((_NS, _PT), jnp.float32),
            pltpu.VMEM((_PT,), jnp.float32),
            pltpu.VMEM_SHARED((_NS, _NPAD), jnp.float32),
        ],
        compiler_params=pltpu.CompilerParams(needs_layout_passes=False),
    )
    return f(dst_r)


def _seg_body(xsa_hbm, xsb_hbm, src_hbm, dst_hbm, outa_hbm, outb_hbm,
              srcv, dstv, rowsv, zerov, acc, sem):
    c = lax.axis_index("c")
    s = lax.axis_index("s")
    zero16 = jnp.zeros((16,), jnp.float32)

    def zb(i, carry):
        for j in range(_DH // 16):
            zerov[i, pl.ds(j * 16, 16)] = zero16
        return carry
    lax.fori_loop(0, _ZR, zb, 0)

    pltpu.sync_copy(src_hbm.at[c, s], srcv)
    pltpu.sync_copy(dst_hbm.at[c, s], dstv)

    rbase = s * _ROWS_T
    for xs_hbm, out_hbm in ((xsa_hbm, outa_hbm), (xsb_hbm, outb_hbm)):
        for t in range(_ROWS_T // _ZR):
            pltpu.sync_copy(zerov, acc.at[pl.ds(rbase + t * _ZR, _ZR)])
        plsc.subcore_barrier()

        def eb(i, carry, xs_hbm=xs_hbm):
            pltpu.async_copy(xs_hbm.at[srcv.at[i]], rowsv, sem).wait()
            pltpu.sync_copy(rowsv, acc.at[dstv.at[i]], add=True)
            return carry
        lax.fori_loop(0, _ECH, eb, 0)

        plsc.subcore_barrier()
        for t in range(_ROWS_T // _ZR):
            pltpu.sync_copy(acc.at[pl.ds(rbase + t * _ZR, _ZR)],
                            out_hbm.at[c, pl.ds(rbase + t * _ZR, _ZR)])


def _seg_call(xsa, xsb, src_r, dst_r):
    out_t = jax.ShapeDtypeStruct((_NC, _NPAD, _DH), jnp.float32)
    f = pl.kernel(
        _seg_body,
        out_type=(out_t, out_t),
        mesh=_sc_mesh(),
        scratch_types=[
            pltpu.VMEM((_ECH, _EK), jnp.int32),
            pltpu.VMEM((_ECH, _EK), jnp.int32),
            pltpu.VMEM((_EK, _DH), jnp.float32),
            pltpu.VMEM((_ZR, _DH), jnp.float32),
            pltpu.VMEM_SHARED((_NPAD, _DH), jnp.float32),
            pltpu.SemaphoreType.DMA,
        ],
        compiler_params=pltpu.CompilerParams(use_tc_tiling_on_sc=False),
    )
    return f(xsa, xsb, src_r, dst_r)


def _lk_body(g_hbm, rsr_hbm, rtg_hbm, idx_hbm, oesr, oetg, orsr, ortg,
             idxv, rowsv, sem):
    c = lax.axis_index("c")
    s = lax.axis_index("s")
    w = c * _NS + s
    pltpu.sync_copy(idx_hbm.at[w], idxv)
    outs = (oesr, oetg, orsr, ortg)
    tabs = (g_hbm, g_hbm, rsr_hbm, rtg_hbm)
    for t in range(4):
        def lb(j, carry, t=t):
            pltpu.async_copy(tabs[t].at[idxv.at[t * _LKC + j]], rowsv,
                             sem).wait()
            pltpu.sync_copy(rowsv, outs[t].at[w, j])
            return carry
        lax.fori_loop(0, _LKC, lb, 0)


def _lk_call(g, rel_sr, rel_tg, idx):
    out_t = jax.ShapeDtypeStruct((_NW, _LKC, _D, _D), jnp.float32)
    f = pl.kernel(
        _lk_body,
        out_type=(out_t, out_t, out_t, out_t),
        mesh=_sc_mesh(),
        scratch_types=[
            pltpu.VMEM((4 * _LKC, _D), jnp.int32),
            pltpu.VMEM((_D, _D), jnp.float32),
            pltpu.SemaphoreType.DMA,
        ],
    )
    return f(g, rel_sr, rel_tg, idx)


def _mm_body(x_ref, w_ref, n_ref, oa_ref, ob_ref):
    r = jnp.dot(x_ref[...], w_ref[...],
                preferred_element_type=jnp.float32) * n_ref[...]
    oa_ref[...] = r[:, :_DH]
    ob_ref[...] = r[:, _DH:]


def _mm_call(x, W, norm2):
    out_t = jax.ShapeDtypeStruct((2 * _N, _DH), jnp.float32)
    hspec = pl.BlockSpec((_LB, _DH), lambda i: (i, 0))
    return pl.pallas_call(
        _mm_body,
        grid=(2 * _N // _LB,),
        in_specs=[pl.BlockSpec((_LB, _D), lambda i: (i, 0)),
                  pl.BlockSpec((_D, _D), lambda i: (0, 0)),
                  pl.BlockSpec((_LB, 1), lambda i: (i, 0))],
        out_specs=(hspec, hspec),
        out_shape=(out_t, out_t),
    )(x, W, norm2)


def _mid_body(ga_ref, gb_ref, xsa_ref, xsb_ref, n_ref, w_ref,
              oa_ref, ob_ref):
    n = n_ref[...]
    g = jnp.concatenate([ga_ref[0], gb_ref[0]], axis=1)
    xs = jnp.concatenate([xsa_ref[...], xsb_ref[...]], axis=1)
    y = jnp.maximum(n * (g + xs), 0.0)
    r = jnp.dot(y, w_ref[...], preferred_element_type=jnp.float32) * n
    oa_ref[...] = r[:, :_DH]
    ob_ref[...] = r[:, _DH:]


def _mid_call(Ga, Gb, xsa, xsb, norm2, W):
    out_t = jax.ShapeDtypeStruct((2 * _N, _DH), jnp.float32)
    gspec = pl.BlockSpec((1, _LB, _DH), lambda g, i: (g, i, 0))
    hspec = pl.BlockSpec((_LB, _DH), lambda g, i: (g * _GB + i, 0))
    return pl.pallas_call(
        _mid_body,
        grid=(_NC, _GB),
        in_specs=[gspec, gspec, hspec, hspec,
                  pl.BlockSpec((_LB, 1), lambda g, i: (g * _GB + i, 0)),
                  pl.BlockSpec((_D, _D), lambda g, i: (0, 0))],
        out_specs=(hspec, hspec),
        out_shape=(out_t, out_t),
    )(Ga, Gb, xsa, xsb, norm2, W)


def _fin_body(ga_ref, gb_ref, xsa_ref, xsb_ref, n_ref, o_ref):
    n = n_ref[...]
    g = jnp.concatenate([ga_ref[0], gb_ref[0]], axis=1)
    xs = jnp.concatenate([xsa_ref[...], xsb_ref[...]], axis=1)
    o_ref[...] = jnp.maximum(n * (g + xs), 0.0)


def _fin_call(Ga, Gb, xsa, xsb, norm2):
    gspec = pl.BlockSpec((1, _LB, _DH), lambda g, i: (g, i, 0))
    hspec = pl.BlockSpec((_LB, _DH), lambda g, i: (g * _GB + i, 0))
    return pl.pallas_call(
        _fin_body,
        grid=(_NC, _GB),
        in_specs=[gspec, gspec, hspec, hspec,
                  pl.BlockSpec((_LB, 1), lambda g, i: (g * _GB + i, 0))],
        out_specs=pl.BlockSpec((_LB, _D), lambda g, i: (g * _GB + i, 0)),
        out_shape=jax.ShapeDtypeStruct((2 * _N, _D), jnp.float32),
    )(Ga, Gb, xsa, xsb, norm2)


def kernel(entity_emb_sr, entity_emb_tg, rel_emb_sr, rel_emb_tg, W0, W1,
           edge_index_sr, edge_index_tg, sr_data, tg_data, sr_rel_data,
           tg_rel_data):
    x = jnp.concatenate([entity_emb_sr, entity_emb_tg], axis=0)
    src = jnp.stack([edge_index_sr[0].astype(jnp.int32),
                     edge_index_tg[0].astype(jnp.int32) + _N])
    dst = jnp.stack([edge_index_sr[1].astype(jnp.int32),
                     edge_index_tg[1].astype(jnp.int32)])
    src_r = src.reshape(_NC, _NS, _ECH, _EK)
    dst_r = dst.reshape(_NC, _NS, _ECH, _EK)

    norm = _norm_call(dst_r)
    norm2 = norm[:, :_N].reshape(2 * _N, 1)

    xs0a, xs0b = _mm_call(x, W0, norm2)
    G0a, G0b = _seg_call(xs0a, xs0b, src_r, dst_r)
    xs1a, xs1b = _mid_call(G0a, G0b, xs0a, xs0b, norm2, W1)
    G1a, G1b = _seg_call(xs1a, xs1b, src_r, dst_r)
    g = _fin_call(G1a, G1b, xs1a, xs1b, norm2)

    def part(a):
        return a.astype(jnp.int32).reshape(_NW, _LKC, _D)
    idx = jnp.concatenate([part(sr_data), part(tg_data) + _N,
                           part(sr_rel_data), part(tg_rel_data)], axis=1)

    oesr, oetg, orsr, ortg = _lk_call(g, rel_emb_sr, rel_emb_tg, idx)
    shp = (_B, _C, _D)
    return (oesr.reshape(shp), oetg.reshape(shp),
            orsr.reshape(shp), ortg.reshape(shp))


# speedup vs baseline: 12.6410x; 1.5448x over previous
import jax
import jax.numpy as jnp
from jax import lax
from jax.experimental import pallas as pl
from jax.experimental.pallas import tpu as pltpu
from jax.experimental.pallas import tpu_sc as plsc

_N = 10000
_D = 128
_E = 320000
_R = 1000
_B = 4096
_C = 25
_NC = 2
_NS = 16
_NW = _NC * _NS

_DH = 32
_NQ = _D // _DH
_NPAD = 10240
_PT = _NPAD // _NS
_EK = 100
_ECH = _E // _NS // _EK
_NB = 8
_LKB = 5
_RT = 624
_ZR = 208
_LB = 1000
_GB = _N // _LB
_LKC = _B * _C // _NW // _D


def _sc_mesh():
    return plsc.VectorSubcoreMesh(
        core_axis_name="c", subcore_axis_name="s",
        num_cores=_NC, num_subcores=_NS)


def _norm_body(dst_hbm, norm_hbm, hist_hbm, histv, dstv, redv, normv):
    c = lax.axis_index("c")
    s = lax.axis_index("s")
    zero16 = jnp.zeros((16,), jnp.float32)
    ones16 = jnp.ones((16,), jnp.float32)

    def zb(i, carry):
        histv[pl.ds(i * 16, 16)] = zero16
        return carry
    lax.fori_loop(0, _NPAD // 16, zb, 0)

    pltpu.sync_copy(dst_hbm.at[c, s], dstv)

    def hb(i, carry):
        idx = dstv[i, pl.ds(0, 16)]
        plsc.addupdate_scatter(histv, [idx], ones16)
        return carry
    lax.fori_loop(0, _E // _NS // 16, hb, 0)

    pltpu.sync_copy(histv, hist_hbm.at[c, s])
    plsc.subcore_barrier()

    base = s * _PT
    for r in range(_NS):
        pltpu.sync_copy(hist_hbm.at[c, r, pl.ds(base, _PT)], redv.at[r])

    def nb(v, carry):
        d = redv[0, pl.ds(v * 16, 16)]
        for r in range(1, _NS):
            d = d + redv[r, pl.ds(v * 16, 16)]
        d = d + 1.0
        bits = plsc.bitcast(d, jnp.int32)
        bits = jnp.int32(0x5F3759DF) - (bits >> 1)
        y = plsc.bitcast(bits, jnp.float32)
        for _ in range(3):
            y = y * (1.5 - 0.5 * d * y * y)
        normv[pl.ds(v * 16, 16)] = y
        return carry
    lax.fori_loop(0, _PT // 16, nb, 0)

    pltpu.sync_copy(normv, norm_hbm.at[c, pl.ds(base, _PT)])


def _norm_call(dst_n):
    f = pl.kernel(
        _norm_body,
        out_type=(jax.ShapeDtypeStruct((_NC, _NPAD), jnp.float32),
                  jax.ShapeDtypeStruct((_NC, _NS, _NPAD), jnp.float32)),
        mesh=_sc_mesh(),
        scratch_types=[
            pltpu.VMEM((_NPAD,), jnp.float32),
            pltpu.VMEM((_E // _NS // 16, 16), jnp.int32),
            pltpu.VMEM((_NS, _PT), jnp.float32),
            pltpu.VMEM((_PT,), jnp.float32),
        ],
        compiler_params=pltpu.CompilerParams(needs_layout_passes=False,
                                             use_tc_tiling_on_sc=False),
    )
    return f(dst_n)[0]


def _seg_body(*args):
    xss = args[:_NQ]
    src_hbm, dst_hbm = args[_NQ], args[_NQ + 1]
    outs = args[_NQ + 2:2 * _NQ + 2]
    srcv, dstv, rowsv, zerov, acc, sem = args[2 * _NQ + 2:]
    c = lax.axis_index("c")
    s = lax.axis_index("s")
    zero16 = jnp.zeros((16,), jnp.float32)

    def zb(i, carry):
        for j in range(_DH // 16):
            zerov[i, pl.ds(j * 16, 16)] = zero16
        return carry
    lax.fori_loop(0, _ZR, zb, 0)

    pltpu.sync_copy(src_hbm.at[c, s], srcv)
    pltpu.sync_copy(dst_hbm.at[c, s], dstv)

    rbase = s * _RT
    last = s == _NS - 1
    for xs_hbm, out_hbm in zip(xss, outs):
        for t in range(3):
            pltpu.sync_copy(zerov, acc.at[pl.ds(rbase + t * _ZR, _ZR)])

        @pl.when(last)
        def _():
            pltpu.sync_copy(zerov.at[pl.ds(0, 16)],
                            acc.at[pl.ds(_N - 16, 16)])
        plsc.subcore_barrier()

        for b in range(_NB):
            pltpu.async_copy(xs_hbm.at[srcv.at[b]], rowsv.at[b], sem.at[b])

        def grp(g0, carry, xs_hbm=xs_hbm):
            base = g0 * _NB
            for b in range(_NB):
                j = base + b
                pltpu.make_async_copy(xs_hbm.at[srcv.at[j]], rowsv.at[b],
                                      sem.at[b]).wait()
                pltpu.sync_copy(rowsv.at[b], acc.at[dstv.at[j]], add=True)
                nxt = j + _NB

                @pl.when(nxt < _ECH)
                def _(b=b, nxt=nxt, xs_hbm=xs_hbm):
                    pltpu.async_copy(xs_hbm.at[srcv.at[nxt]], rowsv.at[b],
                                     sem.at[b])
            return carry
        lax.fori_loop(0, _ECH // _NB, grp, 0)

        plsc.subcore_barrier()
        for t in range(3):
            pltpu.sync_copy(acc.at[pl.ds(rbase + t * _ZR, _ZR)],
                            out_hbm.at[c, pl.ds(rbase + t * _ZR, _ZR)])

        @pl.when(last)
        def _(out_hbm=out_hbm):
            pltpu.sync_copy(acc.at[pl.ds(_N - 16, 16)],
                            out_hbm.at[c, pl.ds(_N - 16, 16)])


def _seg_call(xsq, src_r, dst_r):
    out_t = jax.ShapeDtypeStruct((_NC, _N, _DH), jnp.float32)
    f = pl.kernel(
        _seg_body,
        out_type=(out_t,) * _NQ,
        mesh=_sc_mesh(),
        scratch_types=[
            pltpu.VMEM((_ECH, _EK), jnp.int32),
            pltpu.VMEM((_ECH, _EK), jnp.int32),
            pltpu.VMEM((_NB, _EK, _DH), jnp.float32),
            pltpu.VMEM((_ZR, _DH), jnp.float32),
            pltpu.VMEM_SHARED((_N, _DH), jnp.float32),
            pltpu.SemaphoreType.DMA((_NB,)),
        ],
        compiler_params=pltpu.CompilerParams(use_tc_tiling_on_sc=False),
    )
    return f(*xsq, src_r, dst_r)


def _lk_body(g_hbm, rsr_hbm, rtg_hbm, idx_hbm, oesr, oetg, orsr, ortg,
             idxv, rowsv, sem):
    c = lax.axis_index("c")
    s = lax.axis_index("s")
    w = c * _NS + s
    pltpu.sync_copy(idx_hbm.at[w], idxv)
    outs = (oesr, oetg, orsr, ortg)
    tabs = (g_hbm, g_hbm, rsr_hbm, rtg_hbm)
    for t in range(4):
        for b in range(_LKB):
            pltpu.async_copy(tabs[t].at[idxv.at[t * _LKC + b]], rowsv.at[b],
                             sem.at[b])

        def lg(g0, carry, t=t):
            base = t * _LKC + g0 * _LKB
            for b in range(_LKB):
                j = base + b
                pltpu.make_async_copy(tabs[t].at[idxv.at[j]], rowsv.at[b],
                                      sem.at[b]).wait()
                pltpu.sync_copy(rowsv.at[b], outs[t].at[w, g0 * _LKB + b])
                nxt = j + _LKB

                @pl.when(g0 * _LKB + b + _LKB < _LKC)
                def _(b=b, nxt=nxt, t=t):
                    pltpu.async_copy(tabs[t].at[idxv.at[nxt]], rowsv.at[b],
                                     sem.at[b])
            return carry
        lax.fori_loop(0, _LKC // _LKB, lg, 0)


def _lk_call(g, rel_sr, rel_tg, idx):
    out_t = jax.ShapeDtypeStruct((_NW, _LKC, _D, _D), jnp.float32)
    f = pl.kernel(
        _lk_body,
        out_type=(out_t, out_t, out_t, out_t),
        mesh=_sc_mesh(),
        scratch_types=[
            pltpu.VMEM((4 * _LKC, _D), jnp.int32),
            pltpu.VMEM((_LKB, _D, _D), jnp.float32),
            pltpu.SemaphoreType.DMA((_LKB,)),
        ],
    )
    return f(g, rel_sr, rel_tg, idx)


def _mm_body(x_ref, w_ref, n_ref, *o_refs):
    r = jnp.dot(x_ref[...], w_ref[...],
                preferred_element_type=jnp.float32) * n_ref[...]
    for q in range(_NQ):
        o_refs[q][...] = r[:, q * _DH:(q + 1) * _DH]


def _mm_call(x, W, norm2):
    out_t = jax.ShapeDtypeStruct((2 * _N, _DH), jnp.float32)
    hspec = pl.BlockSpec((_LB, _DH), lambda i: (i, 0))
    return pl.pallas_call(
        _mm_body,
        grid=(2 * _N // _LB,),
        in_specs=[pl.BlockSpec((_LB, _D), lambda i: (i, 0)),
                  pl.BlockSpec((_D, _D), lambda i: (0, 0)),
                  pl.BlockSpec((_LB, 1), lambda i: (i, 0))],
        out_specs=(hspec,) * _NQ,
        out_shape=(out_t,) * _NQ,
    )(x, W, norm2)


def _mid_body(*refs):
    g_refs = refs[:_NQ]
    xs_refs = refs[_NQ:2 * _NQ]
    n_ref, w_ref = refs[2 * _NQ], refs[2 * _NQ + 1]
    o_refs = refs[2 * _NQ + 2:]
    n = n_ref[...]
    g = jnp.concatenate([r[0] for r in g_refs], axis=1)
    xs = jnp.concatenate([r[...] for r in xs_refs], axis=1)
    y = jnp.maximum(n * (g + xs), 0.0)
    r = jnp.dot(y, w_ref[...], preferred_element_type=jnp.float32) * n
    for q in range(_NQ):
        o_refs[q][...] = r[:, q * _DH:(q + 1) * _DH]


def _mid_call(Gq, xsq, norm2, W):
    out_t = jax.ShapeDtypeStruct((2 * _N, _DH), jnp.float32)
    gspec = pl.BlockSpec((1, _LB, _DH), lambda g, i: (g, i, 0))
    hspec = pl.BlockSpec((_LB, _DH), lambda g, i: (g * _GB + i, 0))
    return pl.pallas_call(
        _mid_body,
        grid=(_NC, _GB),
        in_specs=[gspec] * _NQ + [hspec] * _NQ +
                 [pl.BlockSpec((_LB, 1), lambda g, i: (g * _GB + i, 0)),
                  pl.BlockSpec((_D, _D), lambda g, i: (0, 0))],
        out_specs=(hspec,) * _NQ,
        out_shape=(out_t,) * _NQ,
    )(*Gq, *xsq, norm2, W)


def _fin_body(*refs):
    g_refs = refs[:_NQ]
    xs_refs = refs[_NQ:2 * _NQ]
    n_ref, o_ref = refs[2 * _NQ], refs[2 * _NQ + 1]
    g = jnp.concatenate([r[0] for r in g_refs], axis=1)
    xs = jnp.concatenate([r[...] for r in xs_refs], axis=1)
    o_ref[...] = jnp.maximum(n_ref[...] * (g + xs), 0.0)


def _fin_call(Gq, xsq, norm2):
    gspec = pl.BlockSpec((1, _LB, _DH), lambda g, i: (g, i, 0))
    hspec = pl.BlockSpec((_LB, _DH), lambda g, i: (g * _GB + i, 0))
    return pl.pallas_call(
        _fin_body,
        grid=(_NC, _GB),
        in_specs=[gspec] * _NQ + [hspec] * _NQ +
                 [pl.BlockSpec((_LB, 1), lambda g, i: (g * _GB + i, 0))],
        out_specs=pl.BlockSpec((_LB, _D), lambda g, i: (g * _GB + i, 0)),
        out_shape=jax.ShapeDtypeStruct((2 * _N, _D), jnp.float32),
    )(*Gq, *xsq, norm2)


def kernel(entity_emb_sr, entity_emb_tg, rel_emb_sr, rel_emb_tg, W0, W1,
           edge_index_sr, edge_index_tg, sr_data, tg_data, sr_rel_data,
           tg_rel_data):
    x = jnp.concatenate([entity_emb_sr, entity_emb_tg], axis=0)
    src = jnp.stack([edge_index_sr[0].astype(jnp.int32),
                     edge_index_tg[0].astype(jnp.int32) + _N])
    dst = jnp.stack([edge_index_sr[1].astype(jnp.int32),
                     edge_index_tg[1].astype(jnp.int32)])
    src_r = src.reshape(_NC, _NS, _ECH, _EK)
    dst_r = dst.reshape(_NC, _NS, _ECH, _EK)
    dst_n = dst.reshape(_NC, _NS, _E // _NS // 16, 16)

    norm = _norm_call(dst_n)
    norm2 = norm[:, :_N].reshape(2 * _N, 1)

    xs0 = _mm_call(x, W0, norm2)
    G0 = _seg_call(xs0, src_r, dst_r)
    xs1 = _mid_call(G0, xs0, norm2, W1)
    G1 = _seg_call(xs1, src_r, dst_r)
    g = _fin_call(G1, xs1, norm2)

    def part(a):
        return a.astype(jnp.int32).reshape(_NW, _LKC, _D)
    idx = jnp.concatenate([part(sr_data), part(tg_data) + _N,
                           part(sr_rel_data), part(tg_rel_data)], axis=1)

    oesr, oetg, orsr, ortg = _lk_call(g, rel_emb_sr, rel_emb_tg, idx)
    shp = (_B, _C, _D)
    return (oesr.reshape(shp), oetg.reshape(shp),
            orsr.reshape(shp), ortg.reshape(shp))


# speedup vs baseline: 12.7195x; 1.0062x over previous
import jax
import jax.numpy as jnp
from jax import lax
from jax.experimental import pallas as pl
from jax.experimental.pallas import tpu as pltpu
from jax.experimental.pallas import tpu_sc as plsc

_N = 10000
_D = 128
_E = 320000
_R = 1000
_B = 4096
_C = 25
_NC = 2
_NS = 16
_NW = _NC * _NS

_DH = 32
_NQ = _D // _DH
_NPAD = 10240
_PT = _NPAD // _NS
_EK = 100
_ECH = _E // _NS // _EK
_NB = 8
_LKB = 5
_RT = 624
_ZR = 208
_LB = 1000
_GB = _N // _LB
_LKC = _B * _C // _NW // _D


def _sc_mesh():
    return plsc.VectorSubcoreMesh(
        core_axis_name="c", subcore_axis_name="s",
        num_cores=_NC, num_subcores=_NS)


def _norm_body(dst_hbm, norm_hbm, hist_hbm, histv, dstv, redv, normv):
    c = lax.axis_index("c")
    s = lax.axis_index("s")
    zero16 = jnp.zeros((16,), jnp.float32)
    ones16 = jnp.ones((16,), jnp.float32)

    def zb(i, carry):
        histv[pl.ds(i * 16, 16)] = zero16
        return carry
    lax.fori_loop(0, _NPAD // 16, zb, 0)

    pltpu.sync_copy(dst_hbm.at[c, s], dstv)

    def hb(i, carry):
        idx = dstv[i, pl.ds(0, 16)]
        plsc.addupdate_scatter(histv, [idx], ones16)
        return carry
    lax.fori_loop(0, _E // _NS // 16, hb, 0)

    pltpu.sync_copy(histv, hist_hbm.at[c, s])
    plsc.subcore_barrier()

    base = s * _PT
    for r in range(_NS):
        pltpu.sync_copy(hist_hbm.at[c, r, pl.ds(base, _PT)], redv.at[r])

    def nb(v, carry):
        d = redv[0, pl.ds(v * 16, 16)]
        for r in range(1, _NS):
            d = d + redv[r, pl.ds(v * 16, 16)]
        d = d + 1.0
        bits = plsc.bitcast(d, jnp.int32)
        bits = jnp.int32(0x5F3759DF) - (bits >> 1)
        y = plsc.bitcast(bits, jnp.float32)
        for _ in range(3):
            y = y * (1.5 - 0.5 * d * y * y)
        normv[pl.ds(v * 16, 16)] = y
        return carry
    lax.fori_loop(0, _PT // 16, nb, 0)

    pltpu.sync_copy(normv, norm_hbm.at[c, pl.ds(base, _PT)])


def _norm_call(dst_n):
    f = pl.kernel(
        _norm_body,
        out_type=(jax.ShapeDtypeStruct((_NC, _NPAD), jnp.float32),
                  jax.ShapeDtypeStruct((_NC, _NS, _NPAD), jnp.float32)),
        mesh=_sc_mesh(),
        scratch_types=[
            pltpu.VMEM((_NPAD,), jnp.float32),
            pltpu.VMEM((_E // _NS // 16, 16), jnp.int32),
            pltpu.VMEM((_NS, _PT), jnp.float32),
            pltpu.VMEM((_PT,), jnp.float32),
        ],
        compiler_params=pltpu.CompilerParams(needs_layout_passes=False,
                                             use_tc_tiling_on_sc=False),
    )
    return f(dst_n)[0]


def _seg_body(*args):
    xss = args[:_NQ]
    src_hbm, dst_hbm = args[_NQ], args[_NQ + 1]
    outs = args[_NQ + 2:2 * _NQ + 2]
    srcv, dstv, rowsv, zerov, acc, sem = args[2 * _NQ + 2:]
    c = lax.axis_index("c")
    s = lax.axis_index("s")
    zero16 = jnp.zeros((16,), jnp.float32)

    def zb(i, carry):
        for j in range(_DH // 16):
            zerov[i, pl.ds(j * 16, 16)] = zero16
        return carry
    lax.fori_loop(0, _ZR, zb, 0)

    pltpu.sync_copy(src_hbm.at[c, s], srcv)
    pltpu.sync_copy(dst_hbm.at[c, s], dstv)

    rbase = s * _RT
    last = s == _NS - 1
    for xs_hbm, out_hbm in zip(xss, outs):
        for t in range(3):
            pltpu.sync_copy(zerov, acc.at[pl.ds(rbase + t * _ZR, _ZR)])

        @pl.when(last)
        def _():
            pltpu.sync_copy(zerov.at[pl.ds(0, 16)],
                            acc.at[pl.ds(_N - 16, 16)])
        plsc.subcore_barrier()

        for b in range(_NB):
            pltpu.async_copy(xs_hbm.at[srcv.at[b]], rowsv.at[b], sem.at[b])

        def grp(g0, carry, xs_hbm=xs_hbm):
            base = g0 * _NB
            for b in range(_NB):
                j = base + b
                pltpu.make_async_copy(xs_hbm.at[srcv.at[j]], rowsv.at[b],
                                      sem.at[b]).wait()
                pltpu.sync_copy(rowsv.at[b], acc.at[dstv.at[j]], add=True)
                nxt = j + _NB

                @pl.when(nxt < _ECH)
                def _(b=b, nxt=nxt, xs_hbm=xs_hbm):
                    pltpu.async_copy(xs_hbm.at[srcv.at[nxt]], rowsv.at[b],
                                     sem.at[b])
            return carry
        lax.fori_loop(0, _ECH // _NB, grp, 0)

        plsc.subcore_barrier()
        for t in range(3):
            pltpu.sync_copy(acc.at[pl.ds(rbase + t * _ZR, _ZR)],
                            out_hbm.at[c, pl.ds(rbase + t * _ZR, _ZR)])

        @pl.when(last)
        def _(out_hbm=out_hbm):
            pltpu.sync_copy(acc.at[pl.ds(_N - 16, 16)],
                            out_hbm.at[c, pl.ds(_N - 16, 16)])


def _seg_call(xsq, src_r, dst_r):
    out_t = jax.ShapeDtypeStruct((_NC, _N, _DH), jnp.float32)
    f = pl.kernel(
        _seg_body,
        out_type=(out_t,) * _NQ,
        mesh=_sc_mesh(),
        scratch_types=[
            pltpu.VMEM((_ECH, _EK), jnp.int32),
            pltpu.VMEM((_ECH, _EK), jnp.int32),
            pltpu.VMEM((_NB, _EK, _DH), jnp.float32),
            pltpu.VMEM((_ZR, _DH), jnp.float32),
            pltpu.VMEM_SHARED((_N, _DH), jnp.float32),
            pltpu.SemaphoreType.DMA((_NB,)),
        ],
        compiler_params=pltpu.CompilerParams(use_tc_tiling_on_sc=False),
    )
    return f(*xsq, src_r, dst_r)


def _lk_body(g_hbm, rsr_hbm, rtg_hbm, idx_hbm, oesr, oetg, orsr, ortg,
             idxv, rowsv, sem):
    c = lax.axis_index("c")
    s = lax.axis_index("s")
    w = c * _NS + s
    pltpu.sync_copy(idx_hbm.at[w], idxv)
    outs = (oesr, oetg, orsr, ortg)
    tabs = (g_hbm, g_hbm, rsr_hbm, rtg_hbm)
    for t in range(4):
        for b in range(_LKB):
            pltpu.async_copy(tabs[t].at[idxv.at[t * _LKC + b]], rowsv.at[b],
                             sem.at[b])

        def lg(g0, carry, t=t):
            base = t * _LKC + g0 * _LKB
            for b in range(_LKB):
                j = base + b
                pltpu.make_async_copy(tabs[t].at[idxv.at[j]], rowsv.at[b],
                                      sem.at[b]).wait()
                row0 = (w * _LKC + g0 * _LKB + b) * _D
                pltpu.sync_copy(rowsv.at[b], outs[t].at[pl.ds(row0, _D)])
                nxt = j + _LKB

                @pl.when(g0 * _LKB + b + _LKB < _LKC)
                def _(b=b, nxt=nxt, t=t):
                    pltpu.async_copy(tabs[t].at[idxv.at[nxt]], rowsv.at[b],
                                     sem.at[b])
            return carry
        lax.fori_loop(0, _LKC // _LKB, lg, 0)


def _lk_call(g, rel_sr, rel_tg, idx):
    out_t = jax.ShapeDtypeStruct((_B * _C, _D), jnp.float32)
    f = pl.kernel(
        _lk_body,
        out_type=(out_t, out_t, out_t, out_t),
        mesh=_sc_mesh(),
        scratch_types=[
            pltpu.VMEM((4 * _LKC, _D), jnp.int32),
            pltpu.VMEM((_LKB, _D, _D), jnp.float32),
            pltpu.SemaphoreType.DMA((_LKB,)),
        ],
    )
    return f(g, rel_sr, rel_tg, idx)


def _mm_body(x_ref, w_ref, n_ref, *o_refs):
    r = jnp.dot(x_ref[...], w_ref[...],
                preferred_element_type=jnp.float32) * n_ref[...]
    for q in range(_NQ):
        o_refs[q][...] = r[:, q * _DH:(q + 1) * _DH]


def _mm_call(x, W, norm2):
    out_t = jax.ShapeDtypeStruct((2 * _N, _DH), jnp.float32)
    hspec = pl.BlockSpec((_LB, _DH), lambda i: (i, 0))
    return pl.pallas_call(
        _mm_body,
        grid=(2 * _N // _LB,),
        in_specs=[pl.BlockSpec((_LB, _D), lambda i: (i, 0)),
                  pl.BlockSpec((_D, _D), lambda i: (0, 0)),
                  pl.BlockSpec((_LB, 1), lambda i: (i, 0))],
        out_specs=(hspec,) * _NQ,
        out_shape=(out_t,) * _NQ,
    )(x, W, norm2)


def _mid_body(*refs):
    g_refs = refs[:_NQ]
    xs_refs = refs[_NQ:2 * _NQ]
    n_ref, w_ref = refs[2 * _NQ], refs[2 * _NQ + 1]
    o_refs = refs[2 * _NQ + 2:]
    n = n_ref[...]
    g = jnp.concatenate([r[0] for r in g_refs], axis=1)
    xs = jnp.concatenate([r[...] for r in xs_refs], axis=1)
    y = jnp.maximum(n * (g + xs), 0.0)
    r = jnp.dot(y, w_ref[...], preferred_element_type=jnp.float32) * n
    for q in range(_NQ):
        o_refs[q][...] = r[:, q * _DH:(q + 1) * _DH]


def _mid_call(Gq, xsq, norm2, W):
    out_t = jax.ShapeDtypeStruct((2 * _N, _DH), jnp.float32)
    gspec = pl.BlockSpec((1, _LB, _DH), lambda g, i: (g, i, 0))
    hspec = pl.BlockSpec((_LB, _DH), lambda g, i: (g * _GB + i, 0))
    return pl.pallas_call(
        _mid_body,
        grid=(_NC, _GB),
        in_specs=[gspec] * _NQ + [hspec] * _NQ +
                 [pl.BlockSpec((_LB, 1), lambda g, i: (g * _GB + i, 0)),
                  pl.BlockSpec((_D, _D), lambda g, i: (0, 0))],
        out_specs=(hspec,) * _NQ,
        out_shape=(out_t,) * _NQ,
    )(*Gq, *xsq, norm2, W)


def _fin_body(*refs):
    g_refs = refs[:_NQ]
    xs_refs = refs[_NQ:2 * _NQ]
    n_ref, o_ref = refs[2 * _NQ], refs[2 * _NQ + 1]
    g = jnp.concatenate([r[0] for r in g_refs], axis=1)
    xs = jnp.concatenate([r[...] for r in xs_refs], axis=1)
    o_ref[...] = jnp.maximum(n_ref[...] * (g + xs), 0.0)


def _fin_call(Gq, xsq, norm2):
    gspec = pl.BlockSpec((1, _LB, _DH), lambda g, i: (g, i, 0))
    hspec = pl.BlockSpec((_LB, _DH), lambda g, i: (g * _GB + i, 0))
    return pl.pallas_call(
        _fin_body,
        grid=(_NC, _GB),
        in_specs=[gspec] * _NQ + [hspec] * _NQ +
                 [pl.BlockSpec((_LB, 1), lambda g, i: (g * _GB + i, 0))],
        out_specs=pl.BlockSpec((_LB, _D), lambda g, i: (g * _GB + i, 0)),
        out_shape=jax.ShapeDtypeStruct((2 * _N, _D), jnp.float32),
    )(*Gq, *xsq, norm2)


def kernel(entity_emb_sr, entity_emb_tg, rel_emb_sr, rel_emb_tg, W0, W1,
           edge_index_sr, edge_index_tg, sr_data, tg_data, sr_rel_data,
           tg_rel_data):
    x = jnp.concatenate([entity_emb_sr, entity_emb_tg], axis=0)
    src = jnp.stack([edge_index_sr[0].astype(jnp.int32),
                     edge_index_tg[0].astype(jnp.int32) + _N])
    dst = jnp.stack([edge_index_sr[1].astype(jnp.int32),
                     edge_index_tg[1].astype(jnp.int32)])
    src_r = src.reshape(_NC, _NS, _ECH, _EK)
    dst_r = dst.reshape(_NC, _NS, _ECH, _EK)
    dst_n = dst.reshape(_NC, _NS, _E // _NS // 16, 16)

    norm = _norm_call(dst_n)
    norm2 = norm[:, :_N].reshape(2 * _N, 1)

    xs0 = _mm_call(x, W0, norm2)
    G0 = _seg_call(xs0, src_r, dst_r)
    xs1 = _mid_call(G0, xs0, norm2, W1)
    G1 = _seg_call(xs1, src_r, dst_r)
    g = _fin_call(G1, xs1, norm2)

    def part(a):
        return a.astype(jnp.int32).reshape(_NW, _LKC, _D)
    idx = jnp.concatenate([part(sr_data), part(tg_data) + _N,
                           part(sr_rel_data), part(tg_rel_data)], axis=1)

    outs = _lk_call(g, rel_emb_sr, rel_emb_tg, idx)
    return tuple(o.reshape(_B, _C, _D) for o in outs)


# speedup vs baseline: 13.7009x; 1.0772x over previous
import jax
import jax.numpy as jnp
from jax import lax
from jax.experimental import pallas as pl
from jax.experimental.pallas import tpu as pltpu
from jax.experimental.pallas import tpu_sc as plsc

_N = 10000
_D = 128
_E = 320000
_R = 1000
_B = 4096
_C = 25
_NC = 2
_NS = 16
_NW = _NC * _NS

_DH = 32
_NQ = _D // _DH
_NPAD = 10240
_PT = _NPAD // _NS
_EK = 100
_ECH = _E // _NS // _EK
_NB = 8
_LKB = 5
_RT = 624
_ZR = 208
_LB = 1000
_GB = _N // _LB
_LKC = _B * _C // _NW // _D


def _sc_mesh():
    return plsc.VectorSubcoreMesh(
        core_axis_name="c", subcore_axis_name="s",
        num_cores=_NC, num_subcores=_NS)


def _norm_body(dst_hbm, norm_hbm, hist_hbm, histv, dstv, redv, normv):
    c = lax.axis_index("c")
    s = lax.axis_index("s")
    zero16 = jnp.zeros((16,), jnp.float32)
    ones16 = jnp.ones((16,), jnp.float32)

    def zb(i, carry):
        histv[pl.ds(i * 16, 16)] = zero16
        return carry
    lax.fori_loop(0, _NPAD // 16, zb, 0)

    pltpu.sync_copy(dst_hbm.at[c, s], dstv)

    def hb(i, carry):
        idx = dstv[i, pl.ds(0, 16)]
        plsc.addupdate_scatter(histv, [idx], ones16)
        return carry
    lax.fori_loop(0, _E // _NS // 16, hb, 0)

    pltpu.sync_copy(histv, hist_hbm.at[c, s])
    plsc.subcore_barrier()

    base = s * _PT
    for r in range(_NS):
        pltpu.sync_copy(hist_hbm.at[c, r, pl.ds(base, _PT)], redv.at[r])

    def nb(v, carry):
        d = redv[0, pl.ds(v * 16, 16)]
        for r in range(1, _NS):
            d = d + redv[r, pl.ds(v * 16, 16)]
        d = d + 1.0
        bits = plsc.bitcast(d, jnp.int32)
        bits = jnp.int32(0x5F3759DF) - (bits >> 1)
        y = plsc.bitcast(bits, jnp.float32)
        for _ in range(3):
            y = y * (1.5 - 0.5 * d * y * y)
        normv[pl.ds(v * 16, 16)] = y
        return carry
    lax.fori_loop(0, _PT // 16, nb, 0)

    pltpu.sync_copy(normv, norm_hbm.at[c, pl.ds(base, _PT)])


def _norm_call(dst_n):
    f = pl.kernel(
        _norm_body,
        out_type=(jax.ShapeDtypeStruct((_NC, _NPAD), jnp.float32),
                  jax.ShapeDtypeStruct((_NC, _NS, _NPAD), jnp.float32)),
        mesh=_sc_mesh(),
        scratch_types=[
            pltpu.VMEM((_NPAD,), jnp.float32),
            pltpu.VMEM((_E // _NS // 16, 16), jnp.int32),
            pltpu.VMEM((_NS, _PT), jnp.float32),
            pltpu.VMEM((_PT,), jnp.float32),
        ],
        compiler_params=pltpu.CompilerParams(needs_layout_passes=False,
                                             use_tc_tiling_on_sc=False),
    )
    return f(dst_n)[0]


def _seg_body(*args):
    xss = args[:_NQ]
    src_hbm, dst_hbm = args[_NQ], args[_NQ + 1]
    outs = args[_NQ + 2:2 * _NQ + 2]
    srcv, dstv, rowsv, zerov, acc, sem = args[2 * _NQ + 2:]
    c = lax.axis_index("c")
    s = lax.axis_index("s")
    zero16 = jnp.zeros((16,), jnp.float32)

    def zb(i, carry):
        for j in range(_DH // 16):
            zerov[i, pl.ds(j * 16, 16)] = zero16
        return carry
    lax.fori_loop(0, _ZR, zb, 0)

    pltpu.sync_copy(src_hbm.at[c, s], srcv)
    pltpu.sync_copy(dst_hbm.at[c, s], dstv)

    rbase = s * _RT
    last = s == _NS - 1
    for xs_hbm, out_hbm in zip(xss, outs):
        for t in range(3):
            pltpu.sync_copy(zerov, acc.at[pl.ds(rbase + t * _ZR, _ZR)])

        @pl.when(last)
        def _():
            pltpu.sync_copy(zerov.at[pl.ds(0, 16)],
                            acc.at[pl.ds(_N - 16, 16)])
        plsc.subcore_barrier()

        for b in range(_NB):
            pltpu.async_copy(xs_hbm.at[srcv.at[b]], rowsv.at[b], sem.at[b])

        def grp(g0, carry, xs_hbm=xs_hbm):
            base = g0 * _NB
            for b in range(_NB):
                j = base + b
                pltpu.make_async_copy(xs_hbm.at[srcv.at[j]], rowsv.at[b],
                                      sem.at[b]).wait()
                pltpu.sync_copy(rowsv.at[b], acc.at[dstv.at[j]], add=True)
                nxt = j + _NB

                @pl.when(nxt < _ECH)
                def _(b=b, nxt=nxt, xs_hbm=xs_hbm):
                    pltpu.async_copy(xs_hbm.at[srcv.at[nxt]], rowsv.at[b],
                                     sem.at[b])
            return carry
        lax.fori_loop(0, _ECH // _NB, grp, 0)

        plsc.subcore_barrier()
        for t in range(3):
            pltpu.sync_copy(acc.at[pl.ds(rbase + t * _ZR, _ZR)],
                            out_hbm.at[c, pl.ds(rbase + t * _ZR, _ZR)])

        @pl.when(last)
        def _(out_hbm=out_hbm):
            pltpu.sync_copy(acc.at[pl.ds(_N - 16, 16)],
                            out_hbm.at[c, pl.ds(_N - 16, 16)])


def _seg_call(xsq, src_r, dst_r):
    out_t = jax.ShapeDtypeStruct((_NC, _N, _DH), jnp.float32)
    f = pl.kernel(
        _seg_body,
        out_type=(out_t,) * _NQ,
        mesh=_sc_mesh(),
        scratch_types=[
            pltpu.VMEM((_ECH, _EK), jnp.int32),
            pltpu.VMEM((_ECH, _EK), jnp.int32),
            pltpu.VMEM((_NB, _EK, _DH), jnp.float32),
            pltpu.VMEM((_ZR, _DH), jnp.float32),
            pltpu.VMEM_SHARED((_N, _DH), jnp.float32),
            pltpu.SemaphoreType.DMA((_NB,)),
        ],
        compiler_params=pltpu.CompilerParams(use_tc_tiling_on_sc=False),
    )
    return f(*xsq, src_r, dst_r)


_CP = 32
_BT = _B // _NW
_LGR = 8
_NLG = _BT // _LGR


def _lk_gathers(tab, idxv, bufs, b, gi, gsem):
    for k in range(_LGR):
        j = gi * _LGR + k
        ir = idxv.at[j // 4, pl.ds((j % 4) * _CP, _CP)]
        pltpu.async_copy(tab.at[ir], bufs.at[b, k], gsem)


def _lk_body(g_hbm, rsr_hbm, rtg_hbm, idx_hbm, oesr, oetg, orsr, ortg,
             idxv, bufs, gsem, wsem):
    c = lax.axis_index("c")
    s = lax.axis_index("s")
    w = c * _NS + s
    b0 = w * _BT
    outs = (oesr, oetg, orsr, ortg)
    tabs = (g_hbm, g_hbm, rsr_hbm, rtg_hbm)
    for t in range(4):
        pltpu.sync_copy(idx_hbm.at[t, w], idxv)
        _lk_gathers(tabs[t], idxv, bufs, 0, 0, gsem)

        def lg(g2, carry, t=t):
            for b in range(2):
                gi = g2 * 2 + b
                for k in range(_LGR):
                    pltpu.make_async_copy(tabs[t].at[idxv.at[0, pl.ds(0, _CP)]],
                                          bufs.at[b, k], gsem).wait()
                pltpu.async_copy(bufs.at[b, :, pl.ds(0, _C)],
                                 outs[t].at[pl.ds(b0 + gi * _LGR, _LGR)],
                                 wsem.at[b])
                nxt = gi + 1

                @pl.when(nxt < _NLG)
                def _(b=b, nxt=nxt, t=t):
                    bn = 1 - b

                    @pl.when(nxt > 1)
                    def _():
                        pltpu.make_async_copy(
                            bufs.at[bn, :, pl.ds(0, _C)],
                            outs[t].at[pl.ds(b0 + (nxt - 2) * _LGR, _LGR)],
                            wsem.at[bn]).wait()
                    _lk_gathers(tabs[t], idxv, bufs, bn, nxt, gsem)
            return carry
        lax.fori_loop(0, _NLG // 2, lg, 0)
        for b in range(2):
            pltpu.make_async_copy(bufs.at[b, :, pl.ds(0, _C)],
                                  outs[t].at[pl.ds(b0, _LGR)],
                                  wsem.at[b]).wait()


def _lk_call(g, rel_sr, rel_tg, idx):
    out_t = jax.ShapeDtypeStruct((_B, _C, _D), jnp.float32)
    f = pl.kernel(
        _lk_body,
        out_type=(out_t, out_t, out_t, out_t),
        mesh=_sc_mesh(),
        scratch_types=[
            pltpu.VMEM((_CP, _D), jnp.int32),
            pltpu.VMEM((2, _LGR, _CP, _D), jnp.float32),
            pltpu.SemaphoreType.DMA,
            pltpu.SemaphoreType.DMA((2,)),
        ],
    )
    return f(g, rel_sr, rel_tg, idx)


def _mm_body(x_ref, w_ref, n_ref, *o_refs):
    r = jnp.dot(x_ref[...], w_ref[...],
                preferred_element_type=jnp.float32) * n_ref[...]
    for q in range(_NQ):
        o_refs[q][...] = r[:, q * _DH:(q + 1) * _DH]


def _mm_call(x, W, norm2):
    out_t = jax.ShapeDtypeStruct((2 * _N, _DH), jnp.float32)
    hspec = pl.BlockSpec((_LB, _DH), lambda i: (i, 0))
    return pl.pallas_call(
        _mm_body,
        grid=(2 * _N // _LB,),
        in_specs=[pl.BlockSpec((_LB, _D), lambda i: (i, 0)),
                  pl.BlockSpec((_D, _D), lambda i: (0, 0)),
                  pl.BlockSpec((_LB, 1), lambda i: (i, 0))],
        out_specs=(hspec,) * _NQ,
        out_shape=(out_t,) * _NQ,
    )(x, W, norm2)


def _mid_body(*refs):
    g_refs = refs[:_NQ]
    xs_refs = refs[_NQ:2 * _NQ]
    n_ref, w_ref = refs[2 * _NQ], refs[2 * _NQ + 1]
    o_refs = refs[2 * _NQ + 2:]
    n = n_ref[...]
    g = jnp.concatenate([r[0] for r in g_refs], axis=1)
    xs = jnp.concatenate([r[...] for r in xs_refs], axis=1)
    y = jnp.maximum(n * (g + xs), 0.0)
    r = jnp.dot(y, w_ref[...], preferred_element_type=jnp.float32) * n
    for q in range(_NQ):
        o_refs[q][...] = r[:, q * _DH:(q + 1) * _DH]


def _mid_call(Gq, xsq, norm2, W):
    out_t = jax.ShapeDtypeStruct((2 * _N, _DH), jnp.float32)
    gspec = pl.BlockSpec((1, _LB, _DH), lambda g, i: (g, i, 0))
    hspec = pl.BlockSpec((_LB, _DH), lambda g, i: (g * _GB + i, 0))
    return pl.pallas_call(
        _mid_body,
        grid=(_NC, _GB),
        in_specs=[gspec] * _NQ + [hspec] * _NQ +
                 [pl.BlockSpec((_LB, 1), lambda g, i: (g * _GB + i, 0)),
                  pl.BlockSpec((_D, _D), lambda g, i: (0, 0))],
        out_specs=(hspec,) * _NQ,
        out_shape=(out_t,) * _NQ,
    )(*Gq, *xsq, norm2, W)


def _fin_body(*refs):
    g_refs = refs[:_NQ]
    xs_refs = refs[_NQ:2 * _NQ]
    n_ref, o_ref = refs[2 * _NQ], refs[2 * _NQ + 1]
    g = jnp.concatenate([r[0] for r in g_refs], axis=1)
    xs = jnp.concatenate([r[...] for r in xs_refs], axis=1)
    o_ref[...] = jnp.maximum(n_ref[...] * (g + xs), 0.0)


def _fin_call(Gq, xsq, norm2):
    gspec = pl.BlockSpec((1, _LB, _DH), lambda g, i: (g, i, 0))
    hspec = pl.BlockSpec((_LB, _DH), lambda g, i: (g * _GB + i, 0))
    return pl.pallas_call(
        _fin_body,
        grid=(_NC, _GB),
        in_specs=[gspec] * _NQ + [hspec] * _NQ +
                 [pl.BlockSpec((_LB, 1), lambda g, i: (g * _GB + i, 0))],
        out_specs=pl.BlockSpec((_LB, _D), lambda g, i: (g * _GB + i, 0)),
        out_shape=jax.ShapeDtypeStruct((2 * _N, _D), jnp.float32),
    )(*Gq, *xsq, norm2)


def kernel(entity_emb_sr, entity_emb_tg, rel_emb_sr, rel_emb_tg, W0, W1,
           edge_index_sr, edge_index_tg, sr_data, tg_data, sr_rel_data,
           tg_rel_data):
    x = jnp.concatenate([entity_emb_sr, entity_emb_tg], axis=0)
    src = jnp.stack([edge_index_sr[0].astype(jnp.int32),
                     edge_index_tg[0].astype(jnp.int32) + _N])
    dst = jnp.stack([edge_index_sr[1].astype(jnp.int32),
                     edge_index_tg[1].astype(jnp.int32)])
    src_r = src.reshape(_NC, _NS, _ECH, _EK)
    dst_r = dst.reshape(_NC, _NS, _ECH, _EK)
    dst_n = dst.reshape(_NC, _NS, _E // _NS // 16, 16)

    norm = _norm_call(dst_n)
    norm2 = norm[:, :_N].reshape(2 * _N, 1)

    xs0 = _mm_call(x, W0, norm2)
    G0 = _seg_call(xs0, src_r, dst_r)
    xs1 = _mid_call(G0, xs0, norm2, W1)
    G1 = _seg_call(xs1, src_r, dst_r)
    g = _fin_call(G1, xs1, norm2)

    def padidx(a, shift):
        a = a.astype(jnp.int32) + shift
        pad = jnp.broadcast_to(a[:, :1], (_B, _CP - _C))
        return jnp.concatenate([a, pad], axis=1)
    idx = jnp.stack([padidx(sr_data, 0), padidx(tg_data, _N),
                     padidx(sr_rel_data, 0), padidx(tg_rel_data, 0)])
    idx = idx.reshape(4, _NW, _CP, _D)

    return _lk_call(g, rel_emb_sr, rel_emb_tg, idx)


# speedup vs baseline: 15.3834x; 1.1228x over previous
import jax
import jax.numpy as jnp
from jax import lax
from jax.experimental import pallas as pl
from jax.experimental.pallas import tpu as pltpu
from jax.experimental.pallas import tpu_sc as plsc

_N = 10000
_D = 128
_E = 320000
_R = 1000
_B = 4096
_C = 25
_NC = 2
_NS = 16
_NW = _NC * _NS

_DH = 32
_NQ = _D // _DH
_NPAD = 10240
_PT = _NPAD // _NS
_EK = 100
_ECH = _E // _NS // _EK
_NB = 8
_LKB = 5
_RT = 624
_ZR = 208
_LB = 1000
_GB = _N // _LB
_LKC = _B * _C // _NW // _D


def _sc_mesh():
    return plsc.VectorSubcoreMesh(
        core_axis_name="c", subcore_axis_name="s",
        num_cores=_NC, num_subcores=_NS)


def _norm_body(dst_hbm, norm_hbm, hist_hbm, histv, dstv, redv, normv):
    c = lax.axis_index("c")
    s = lax.axis_index("s")
    zero16 = jnp.zeros((16,), jnp.float32)
    ones16 = jnp.ones((16,), jnp.float32)

    def zb(i, carry):
        histv[pl.ds(i * 16, 16)] = zero16
        return carry
    lax.fori_loop(0, _NPAD // 16, zb, 0)

    pltpu.sync_copy(dst_hbm.at[c, s], dstv)

    def hb(i, carry):
        idx = dstv[i, pl.ds(0, 16)]
        plsc.addupdate_scatter(histv, [idx], ones16)
        return carry
    lax.fori_loop(0, _E // _NS // 16, hb, 0)

    pltpu.sync_copy(histv, hist_hbm.at[c, s])
    plsc.subcore_barrier()

    base = s * _PT
    for r in range(_NS):
        pltpu.sync_copy(hist_hbm.at[c, r, pl.ds(base, _PT)], redv.at[r])

    def nb(v, carry):
        d = redv[0, pl.ds(v * 16, 16)]
        for r in range(1, _NS):
            d = d + redv[r, pl.ds(v * 16, 16)]
        d = d + 1.0
        bits = plsc.bitcast(d, jnp.int32)
        bits = jnp.int32(0x5F3759DF) - (bits >> 1)
        y = plsc.bitcast(bits, jnp.float32)
        for _ in range(3):
            y = y * (1.5 - 0.5 * d * y * y)
        normv[pl.ds(v * 16, 16)] = y
        return carry
    lax.fori_loop(0, _PT // 16, nb, 0)

    pltpu.sync_copy(normv, norm_hbm.at[c, pl.ds(base, _PT)])


def _norm_call(dst_n):
    f = pl.kernel(
        _norm_body,
        out_type=(jax.ShapeDtypeStruct((_NC, _NPAD), jnp.float32),
                  jax.ShapeDtypeStruct((_NC, _NS, _NPAD), jnp.float32)),
        mesh=_sc_mesh(),
        scratch_types=[
            pltpu.VMEM((_NPAD,), jnp.float32),
            pltpu.VMEM((_E // _NS // 16, 16), jnp.int32),
            pltpu.VMEM((_NS, _PT), jnp.float32),
            pltpu.VMEM((_PT,), jnp.float32),
        ],
        compiler_params=pltpu.CompilerParams(needs_layout_passes=False,
                                             use_tc_tiling_on_sc=False),
    )
    return f(dst_n)[0]


def _seg_body(xs_hbm, src_hbm, dst_hbm, out_hbm,
              srcv, dstv, rowsv, zerov, acc, sem):
    c = lax.axis_index("c")
    s = lax.axis_index("s")
    zero16 = jnp.zeros((16,), jnp.float32)

    def zb(i, carry):
        for j in range(_DH // 16):
            zerov[i, pl.ds(j * 16, 16)] = zero16
        return carry
    lax.fori_loop(0, _ZR, zb, 0)

    pltpu.sync_copy(dst_hbm.at[c, s], dstv)

    rbase = s * _RT
    last = s == _NS - 1
    for q in range(_NQ):
        tab = xs_hbm
        pltpu.sync_copy(src_hbm.at[q, c, s], srcv)
        for t in range(3):
            pltpu.sync_copy(zerov, acc.at[pl.ds(rbase + t * _ZR, _ZR)])

        @pl.when(last)
        def _():
            pltpu.sync_copy(zerov.at[pl.ds(0, 16)],
                            acc.at[pl.ds(_N - 16, 16)])
        plsc.subcore_barrier()

        for b in range(_NB):
            pltpu.async_copy(tab.at[srcv.at[b]], rowsv.at[b], sem.at[b])

        def grp(g0, carry, tab=tab):
            base = g0 * _NB
            for b in range(_NB):
                j = base + b
                pltpu.make_async_copy(tab.at[srcv.at[j]], rowsv.at[b],
                                      sem.at[b]).wait()
                pltpu.sync_copy(rowsv.at[b], acc.at[dstv.at[j]], add=True)
                nxt = j + _NB

                @pl.when(nxt < _ECH)
                def _(b=b, nxt=nxt, tab=tab):
                    pltpu.async_copy(tab.at[srcv.at[nxt]], rowsv.at[b],
                                     sem.at[b])
            return carry
        lax.fori_loop(0, _ECH // _NB, grp, 0)

        plsc.subcore_barrier()
        for t in range(3):
            pltpu.sync_copy(
                acc.at[pl.ds(rbase + t * _ZR, _ZR)],
                out_hbm.at[c, pl.ds(rbase + t * _ZR, _ZR),
                           pl.ds(q * _DH, _DH)])

        @pl.when(last)
        def _(q=q):
            pltpu.sync_copy(
                acc.at[pl.ds(_N - 16, 16)],
                out_hbm.at[c, pl.ds(_N - 16, 16), pl.ds(q * _DH, _DH)])


def _seg_call(xs, src_q, dst_r):
    f = pl.kernel(
        _seg_body,
        out_type=jax.ShapeDtypeStruct((_NC, _N, _D), jnp.float32),
        mesh=_sc_mesh(),
        scratch_types=[
            pltpu.VMEM((_ECH, _EK), jnp.int32),
            pltpu.VMEM((_ECH, _EK), jnp.int32),
            pltpu.VMEM((_NB, _EK, _DH), jnp.float32),
            pltpu.VMEM((_ZR, _DH), jnp.float32),
            pltpu.VMEM_SHARED((_N, _DH), jnp.float32),
            pltpu.SemaphoreType.DMA((_NB,)),
        ],
        compiler_params=pltpu.CompilerParams(use_tc_tiling_on_sc=False),
    )
    return f(xs.reshape(_NQ * 2 * _N, _DH), src_q, dst_r)


_CP = 32
_BT = _B // _NW
_LGR = 8
_NLG = _BT // _LGR


def _lk_gathers(tab, idxv, bufs, b, gi, gsem):
    for k in range(_LGR):
        j = gi * _LGR + k
        ir = idxv.at[j // 4, pl.ds((j % 4) * _CP, _CP)]
        pltpu.async_copy(tab.at[ir], bufs.at[b, k], gsem)


def _lk_body(g_hbm, rsr_hbm, rtg_hbm, idx_hbm, oesr, oetg, orsr, ortg,
             idxv, bufs, gsem, wsem):
    c = lax.axis_index("c")
    s = lax.axis_index("s")
    w = c * _NS + s
    b0 = w * _BT
    outs = (oesr, oetg, orsr, ortg)
    tabs = (g_hbm, g_hbm, rsr_hbm, rtg_hbm)
    for t in range(4):
        pltpu.sync_copy(idx_hbm.at[t, w], idxv)
        _lk_gathers(tabs[t], idxv, bufs, 0, 0, gsem)

        def lg(g2, carry, t=t):
            for b in range(2):
                gi = g2 * 2 + b
                for k in range(_LGR):
                    pltpu.make_async_copy(tabs[t].at[idxv.at[0, pl.ds(0, _CP)]],
                                          bufs.at[b, k], gsem).wait()
                pltpu.async_copy(bufs.at[b, :, pl.ds(0, _C)],
                                 outs[t].at[pl.ds(b0 + gi * _LGR, _LGR)],
                                 wsem.at[b])
                nxt = gi + 1

                @pl.when(nxt < _NLG)
                def _(b=b, nxt=nxt, t=t):
                    bn = 1 - b

                    @pl.when(nxt > 1)
                    def _():
                        pltpu.make_async_copy(
                            bufs.at[bn, :, pl.ds(0, _C)],
                            outs[t].at[pl.ds(b0 + (nxt - 2) * _LGR, _LGR)],
                            wsem.at[bn]).wait()
                    _lk_gathers(tabs[t], idxv, bufs, bn, nxt, gsem)
            return carry
        lax.fori_loop(0, _NLG // 2, lg, 0)
        for b in range(2):
            pltpu.make_async_copy(bufs.at[b, :, pl.ds(0, _C)],
                                  outs[t].at[pl.ds(b0, _LGR)],
                                  wsem.at[b]).wait()


def _lk_call(g, rel_sr, rel_tg, idx):
    out_t = jax.ShapeDtypeStruct((_B, _C, _D), jnp.float32)
    f = pl.kernel(
        _lk_body,
        out_type=(out_t, out_t, out_t, out_t),
        mesh=_sc_mesh(),
        scratch_types=[
            pltpu.VMEM((_CP, _D), jnp.int32),
            pltpu.VMEM((2, _LGR, _CP, _D), jnp.float32),
            pltpu.SemaphoreType.DMA,
            pltpu.SemaphoreType.DMA((2,)),
        ],
    )
    return f(g, rel_sr, rel_tg, idx)


def _mm_body(x_ref, w_ref, n_ref, o_ref):
    o_ref[...] = jnp.dot(x_ref[...], w_ref[...],
                         preferred_element_type=jnp.float32) * n_ref[...]


def _mm_call(x, W, norm2):
    return pl.pallas_call(
        _mm_body,
        grid=(2 * _N // _LB,),
        in_specs=[pl.BlockSpec((_LB, _D), lambda i: (i, 0)),
                  pl.BlockSpec((_D, _D), lambda i: (0, 0)),
                  pl.BlockSpec((_LB, 1), lambda i: (i, 0))],
        out_specs=pl.BlockSpec((_LB, _D), lambda i: (i, 0)),
        out_shape=jax.ShapeDtypeStruct((2 * _N, _D), jnp.float32),
    )(x, W, norm2)


def _mid_body(g_ref, xs_ref, n_ref, w_ref, o_ref):
    n = n_ref[...]
    y = jnp.maximum(n * (g_ref[0] + xs_ref[...]), 0.0)
    o_ref[...] = jnp.dot(y, w_ref[...],
                         preferred_element_type=jnp.float32) * n


def _mid_call(G, xs, norm2, W):
    gspec = pl.BlockSpec((1, _LB, _D), lambda g, i: (g, i, 0))
    hspec = pl.BlockSpec((_LB, _D), lambda g, i: (g * _GB + i, 0))
    return pl.pallas_call(
        _mid_body,
        grid=(_NC, _GB),
        in_specs=[gspec, hspec,
                  pl.BlockSpec((_LB, 1), lambda g, i: (g * _GB + i, 0)),
                  pl.BlockSpec((_D, _D), lambda g, i: (0, 0))],
        out_specs=hspec,
        out_shape=jax.ShapeDtypeStruct((2 * _N, _D), jnp.float32),
    )(G, xs, norm2, W)


def _fin_body(g_ref, xs_ref, n_ref, o_ref):
    o_ref[...] = jnp.maximum(n_ref[...] * (g_ref[0] + xs_ref[...]), 0.0)


def _fin_call(G, xs, norm2):
    gspec = pl.BlockSpec((1, _LB, _D), lambda g, i: (g, i, 0))
    hspec = pl.BlockSpec((_LB, _D), lambda g, i: (g * _GB + i, 0))
    return pl.pallas_call(
        _fin_body,
        grid=(_NC, _GB),
        in_specs=[gspec, hspec,
                  pl.BlockSpec((_LB, 1), lambda g, i: (g * _GB + i, 0))],
        out_specs=pl.BlockSpec((_LB, _D), lambda g, i: (g * _GB + i, 0)),
        out_shape=jax.ShapeDtypeStruct((2 * _N, _D), jnp.float32),
    )(G, xs, norm2)


def kernel(entity_emb_sr, entity_emb_tg, rel_emb_sr, rel_emb_tg, W0, W1,
           edge_index_sr, edge_index_tg, sr_data, tg_data, sr_rel_data,
           tg_rel_data):
    x = jnp.concatenate([entity_emb_sr, entity_emb_tg], axis=0)
    src = jnp.stack([edge_index_sr[0].astype(jnp.int32),
                     edge_index_tg[0].astype(jnp.int32) + _N])
    dst = jnp.stack([edge_index_sr[1].astype(jnp.int32),
                     edge_index_tg[1].astype(jnp.int32)])
    src_q = (src[None] * _NQ +
             jnp.arange(_NQ, dtype=jnp.int32)[:, None, None])
    src_q = src_q.reshape(_NQ, _NC, _NS, _ECH, _EK)
    dst_r = dst.reshape(_NC, _NS, _ECH, _EK)
    dst_n = dst.reshape(_NC, _NS, _E // _NS // 16, 16)

    norm = _norm_call(dst_n)
    norm2 = norm[:, :_N].reshape(2 * _N, 1)

    xs0 = _mm_call(x, W0, norm2)
    G0 = _seg_call(xs0, src_q, dst_r)
    xs1 = _mid_call(G0, xs0, norm2, W1)
    G1 = _seg_call(xs1, src_q, dst_r)
    g = _fin_call(G1, xs1, norm2)

    def padidx(a, shift):
        a = a.astype(jnp.int32) + shift
        pad = jnp.broadcast_to(a[:, :1], (_B, _CP - _C))
        return jnp.concatenate([a, pad], axis=1)
    idx = jnp.stack([padidx(sr_data, 0), padidx(tg_data, _N),
                     padidx(sr_rel_data, 0), padidx(tg_rel_data, 0)])
    idx = idx.reshape(4, _NW, _CP, _D)

    return _lk_call(g, rel_emb_sr, rel_emb_tg, idx)
